# baseline (speedup 1.0000x reference)
import functools

import jax
import jax.numpy as jnp
from jax.experimental import pallas as pl
from jax.experimental.pallas import tpu as pltpu

_TILE = 256
_MAX_OPS_STATIC = 4


def _fused_body(x_ref, W_in_ref, b_in_ref, W_symf_ref, b_symf_ref, W_q_ref,
                w_sent_ref, Wgs_ref, bgs_ref, Wsh_ref, hopmask_ref,
                out_ref, pidx_ref, symf_ref, *, num_nodes, sym_dim):
    T = x_ref.shape[0]
    K = num_nodes
    S = sym_dim
    f32 = jnp.float32

    z = jnp.dot(x_ref[...], W_in_ref[...], preferred_element_type=f32)
    z = z + b_in_ref[...]

    sym_flat = jnp.tanh(
        jnp.dot(z, W_symf_ref[...], preferred_element_type=f32)
        + b_symf_ref[...])
    symf_ref[...] = sym_flat

    q = jnp.dot(z, W_q_ref[...], preferred_element_type=f32)
    node_logits = [
        jnp.sum(sym_flat[:, k * S:(k + 1) * S] * q, axis=1, keepdims=True)
        for k in range(K)
    ]
    sent = jnp.dot(z, w_sent_ref[...], preferred_element_type=f32)

    neg_inf = float("-inf")
    pad = jnp.full((T, 16 - (K + 1)), neg_inf, dtype=f32)
    vals = jnp.concatenate(node_logits + [sent, pad], axis=1)
    col = jax.lax.broadcasted_iota(jnp.int32, (T, 16), 1)

    idx_cols = []
    for _ in range(_MAX_OPS_STATIC):
        m = jnp.max(vals, axis=1, keepdims=True)
        first = jnp.min(jnp.where(vals == m, col, 16), axis=1, keepdims=True)
        idx_cols.append(first)
        vals = jnp.where(col == first, neg_inf, vals)
    pidx_ref[...] = jnp.concatenate(idx_cols, axis=1)

    Wg = Wgs_ref[...]
    Ws = Wsh_ref[...]
    bg = bgs_ref[...]
    sym_b = sym_flat.astype(jnp.bfloat16)
    kcol = jax.lax.broadcasted_iota(jnp.int32, (T, K * S), 1) // S
    out = z
    done = jnp.zeros((T, 1), dtype=jnp.bool_)
    for hop in range(_MAX_OPS_STATIC):
        hidx = idx_cols[hop]
        is_sent = hidx == K
        hop_live = hopmask_ref[hop] != 0
        active = jnp.logical_not(done) & jnp.logical_not(is_sent) & hop_live
        done = done | is_sent
        msym = jnp.where(kcol == hidx, sym_b, jnp.bfloat16(0.0))
        onehot = jnp.where(col == hidx, 1.0, 0.0).astype(f32)
        gl = (jnp.dot(msym, Wg, preferred_element_type=f32)
              + jnp.dot(onehot, bg, preferred_element_type=f32))
        hl = jnp.dot(msym, Ws, preferred_element_type=f32)
        new_out = out * jax.nn.sigmoid(gl) + jnp.tanh(hl)
        out = jnp.where(active, new_out, out)
    out_ref[...] = out


def kernel(x, W_in, b_in, W_sym, b_sym, W_q, w_sent, W_gate, b_gate, W_shift,
           max_ops):
    N, D_in = x.shape
    K, D_lat, S = W_sym.shape
    D = W_in.shape[1]

    W_symf = jnp.transpose(W_sym, (1, 0, 2)).reshape(D_lat, K * S)
    b_symf = b_sym.reshape(1, K * S)
    b_in2 = b_in.reshape(1, D)
    w_sent2 = w_sent.reshape(D_lat, 1)
    Wg_flat = W_gate.reshape(K * S, D).astype(jnp.bfloat16)
    Ws_flat = W_shift.reshape(K * S, D).astype(jnp.bfloat16)
    bg_pad = jnp.zeros((16, D), dtype=b_gate.dtype).at[:K].set(b_gate)
    hopmask = (jnp.arange(_MAX_OPS_STATIC, dtype=jnp.int32)
               < jnp.asarray(max_ops, jnp.int32)).astype(jnp.int32)

    grid = (N // _TILE,)
    body = functools.partial(_fused_body, num_nodes=K, sym_dim=S)
    out, pidx, sym_flat = pl.pallas_call(
        body,
        grid=grid,
        in_specs=[
            pl.BlockSpec((_TILE, D_in), lambda i: (i, 0)),
            pl.BlockSpec((D_in, D), lambda i: (0, 0)),
            pl.BlockSpec((1, D), lambda i: (0, 0)),
            pl.BlockSpec((D_lat, K * S), lambda i: (0, 0)),
            pl.BlockSpec((1, K * S), lambda i: (0, 0)),
            pl.BlockSpec((D_lat, S), lambda i: (0, 0)),
            pl.BlockSpec((D_lat, 1), lambda i: (0, 0)),
            pl.BlockSpec((K * S, D), lambda i: (0, 0)),
            pl.BlockSpec((16, D), lambda i: (0, 0)),
            pl.BlockSpec((K * S, D), lambda i: (0, 0)),
            pl.BlockSpec(memory_space=pltpu.SMEM),
        ],
        out_specs=[
            pl.BlockSpec((_TILE, D), lambda i: (i, 0)),
            pl.BlockSpec((_TILE, _MAX_OPS_STATIC), lambda i: (i, 0)),
            pl.BlockSpec((_TILE, K * S), lambda i: (i, 0)),
        ],
        out_shape=[
            jax.ShapeDtypeStruct((N, D), jnp.float32),
            jax.ShapeDtypeStruct((N, _MAX_OPS_STATIC), jnp.int32),
            jax.ShapeDtypeStruct((N, K * S), jnp.float32),
        ],
    )(x, W_in, b_in2, W_symf, b_symf, W_q, w_sent2, Wg_flat, bg_pad, Ws_flat,
      hopmask)
    return out, pidx, sym_flat.reshape(N, K, S)


# speedup vs baseline: 1.1624x; 1.1624x over previous
import functools

import jax
import jax.numpy as jnp
from jax.experimental import pallas as pl
from jax.experimental.pallas import tpu as pltpu

_TILE = 256
_MAX_OPS_STATIC = 4


def _fused_body(x_ref, W_in_ref, b_in_ref, W_symf_ref, b_symf_ref, W_q_ref,
                w_sent_ref, Wgs_ref, bgs_ref, Wsh_ref, hopmask_ref,
                out_ref, pidx_ref, symf_ref, zsc_ref, ssc_ref,
                *, num_nodes, sym_dim):
    T = out_ref.shape[0]
    K = num_nodes
    S = sym_dim
    f32 = jnp.float32

    i = pl.program_id(0)
    ph = jax.lax.rem(i, 2)
    pp = 1 - ph

    z = zsc_ref[pp]
    sym_flat = ssc_ref[pp]
    symf_ref[...] = sym_flat

    q = jnp.dot(z, W_q_ref[...], preferred_element_type=f32)
    sent = jnp.dot(z, w_sent_ref[...], preferred_element_type=f32)

    zn = jnp.dot(x_ref[...], W_in_ref[...], preferred_element_type=f32)
    zn = zn + b_in_ref[...]

    node_logits = [
        jnp.sum(sym_flat[:, k * S:(k + 1) * S] * q, axis=1, keepdims=True)
        for k in range(K)
    ]

    symn = jnp.tanh(
        jnp.dot(zn, W_symf_ref[...], preferred_element_type=f32)
        + b_symf_ref[...])

    neg_inf = float("-inf")
    pad = jnp.full((T, 16 - (K + 1)), neg_inf, dtype=f32)
    vals = jnp.concatenate(node_logits + [sent, pad], axis=1)
    col = jax.lax.broadcasted_iota(jnp.int32, (T, 16), 1)

    idx_cols = []
    for _ in range(_MAX_OPS_STATIC):
        m = jnp.max(vals, axis=1, keepdims=True)
        first = jnp.min(jnp.where(vals == m, col, 16), axis=1, keepdims=True)
        idx_cols.append(first)
        vals = jnp.where(col == first, neg_inf, vals)
    pidx_ref[...] = jnp.concatenate(idx_cols, axis=1)

    zsc_ref[ph] = zn
    ssc_ref[ph] = symn

    Wg = Wgs_ref[...]
    Ws = Wsh_ref[...]
    bg = bgs_ref[...]
    kcol = jax.lax.broadcasted_iota(jnp.int32, (T, K * S), 1) // S
    out = z
    done = jnp.zeros((T, 1), dtype=jnp.bool_)
    for hop in range(_MAX_OPS_STATIC):
        hidx = idx_cols[hop]
        is_sent = hidx == K
        hop_live = hopmask_ref[hop] != 0
        active = jnp.logical_not(done) & jnp.logical_not(is_sent) & hop_live
        done = done | is_sent
        msym = jnp.where(kcol == hidx, sym_flat, 0.0)
        onehot = jnp.where(col == hidx, 1.0, 0.0).astype(f32)
        gl = (jnp.dot(msym, Wg, preferred_element_type=f32)
              + jnp.dot(onehot, bg, preferred_element_type=f32))
        hl = jnp.dot(msym, Ws, preferred_element_type=f32)
        new_out = out * jax.nn.sigmoid(gl) + jnp.tanh(hl)
        out = jnp.where(active, new_out, out)
    out_ref[...] = out


def kernel(x, W_in, b_in, W_sym, b_sym, W_q, w_sent, W_gate, b_gate, W_shift,
           max_ops):
    N, D_in = x.shape
    K, D_lat, S = W_sym.shape
    D = W_in.shape[1]
    NT = N // _TILE

    W_symf = jnp.transpose(W_sym, (1, 0, 2)).reshape(D_lat, K * S)
    b_symf = b_sym.reshape(1, K * S)
    b_in2 = b_in.reshape(1, D)
    w_sent2 = w_sent.reshape(D_lat, 1)
    Wg_flat = W_gate.reshape(K * S, D)
    Ws_flat = W_shift.reshape(K * S, D)
    bg_pad = jnp.zeros((16, D), dtype=b_gate.dtype).at[:K].set(b_gate)
    hopmask = (jnp.arange(_MAX_OPS_STATIC, dtype=jnp.int32)
               < jnp.asarray(max_ops, jnp.int32)).astype(jnp.int32)

    grid = (NT + 1,)
    body = functools.partial(_fused_body, num_nodes=K, sym_dim=S)
    prev = lambda i: (jnp.maximum(i - 1, 0), 0)
    cur = lambda i: (jnp.minimum(i, NT - 1), 0)
    out, pidx, sym_flat = pl.pallas_call(
        body,
        grid=grid,
        in_specs=[
            pl.BlockSpec((_TILE, D_in), cur),
            pl.BlockSpec((D_in, D), lambda i: (0, 0)),
            pl.BlockSpec((1, D), lambda i: (0, 0)),
            pl.BlockSpec((D_lat, K * S), lambda i: (0, 0)),
            pl.BlockSpec((1, K * S), lambda i: (0, 0)),
            pl.BlockSpec((D_lat, S), lambda i: (0, 0)),
            pl.BlockSpec((D_lat, 1), lambda i: (0, 0)),
            pl.BlockSpec((K * S, D), lambda i: (0, 0)),
            pl.BlockSpec((16, D), lambda i: (0, 0)),
            pl.BlockSpec((K * S, D), lambda i: (0, 0)),
            pl.BlockSpec(memory_space=pltpu.SMEM),
        ],
        out_specs=[
            pl.BlockSpec((_TILE, D), prev),
            pl.BlockSpec((_TILE, _MAX_OPS_STATIC), prev),
            pl.BlockSpec((_TILE, K * S), prev),
        ],
        out_shape=[
            jax.ShapeDtypeStruct((N, D), jnp.float32),
            jax.ShapeDtypeStruct((N, _MAX_OPS_STATIC), jnp.int32),
            jax.ShapeDtypeStruct((N, K * S), jnp.float32),
        ],
        scratch_shapes=[
            pltpu.VMEM((2, _TILE, D), jnp.float32),
            pltpu.VMEM((2, _TILE, K * S), jnp.float32),
        ],
    )(x, W_in, b_in2, W_symf, b_symf, W_q, w_sent2, Wg_flat, bg_pad, Ws_flat,
      hopmask)
    return out, pidx, sym_flat.reshape(N, K, S)


# speedup vs baseline: 1.1634x; 1.0008x over previous
import functools

import jax
import jax.numpy as jnp
from jax.experimental import pallas as pl
from jax.experimental.pallas import tpu as pltpu

_TILE = 256
_MAX_OPS_STATIC = 4


def _fused_body(x_ref, W_in_ref, b_in_ref, W_symf_ref, b_symf_ref, W_q_ref,
                w_sent_ref, Wgs_ref, bgs_ref, Wsh_ref, hopmask_ref,
                out_ref, pidx_ref, symf_ref, zsc_ref, ssc_ref,
                *, num_nodes, sym_dim):
    T = out_ref.shape[0]
    K = num_nodes
    S = sym_dim
    f32 = jnp.float32

    i = pl.program_id(0)
    ph = jax.lax.rem(i, 2)
    pp = 1 - ph

    z = zsc_ref[pp]
    sym_flat = ssc_ref[pp]

    q = jnp.dot(z, W_q_ref[...], preferred_element_type=f32)
    sent = jnp.dot(z, w_sent_ref[...], preferred_element_type=f32)

    zn = jnp.dot(x_ref[...], W_in_ref[...], preferred_element_type=f32)
    zn = zn + b_in_ref[...]

    node_logits = [
        jnp.sum(sym_flat[:, k * S:(k + 1) * S] * q, axis=1, keepdims=True)
        for k in range(K)
    ]

    symn = jnp.tanh(
        jnp.dot(zn, W_symf_ref[...], preferred_element_type=f32)
        + b_symf_ref[...])
    symf_ref[...] = symn

    neg_inf = float("-inf")
    pad = jnp.full((T, 16 - (K + 1)), neg_inf, dtype=f32)
    vals = jnp.concatenate(node_logits + [sent, pad], axis=1)
    col = jax.lax.broadcasted_iota(jnp.int32, (T, 16), 1)

    idx_cols = []
    for _ in range(_MAX_OPS_STATIC):
        m = jnp.max(vals, axis=1, keepdims=True)
        first = jnp.min(jnp.where(vals == m, col, 16), axis=1, keepdims=True)
        idx_cols.append(first)
        vals = jnp.where(col == first, neg_inf, vals)
    pidx_ref[...] = jnp.concatenate(idx_cols, axis=1)

    zsc_ref[ph] = zn
    ssc_ref[ph] = symn

    Wg = Wgs_ref[...]
    Ws = Wsh_ref[...]
    bg = bgs_ref[...]
    kcol = jax.lax.broadcasted_iota(jnp.int32, (T, K * S), 1) // S
    out = z
    done = jnp.zeros((T, 1), dtype=jnp.bool_)
    for hop in range(_MAX_OPS_STATIC):
        hidx = idx_cols[hop]
        is_sent = hidx == K
        hop_live = hopmask_ref[hop] != 0
        active = jnp.logical_not(done) & jnp.logical_not(is_sent) & hop_live
        done = done | is_sent
        msym = jnp.where(kcol == hidx, sym_flat, 0.0)
        onehot = jnp.where(col == hidx, 1.0, 0.0).astype(f32)
        gl = (jnp.dot(msym, Wg, preferred_element_type=f32)
              + jnp.dot(onehot, bg, preferred_element_type=f32))
        hl = jnp.dot(msym, Ws, preferred_element_type=f32)
        new_out = out * jax.nn.sigmoid(gl) + jnp.tanh(hl)
        out = jnp.where(active, new_out, out)
    out_ref[...] = out


def kernel(x, W_in, b_in, W_sym, b_sym, W_q, w_sent, W_gate, b_gate, W_shift,
           max_ops):
    N, D_in = x.shape
    K, D_lat, S = W_sym.shape
    D = W_in.shape[1]
    NT = N // _TILE

    W_symf = jnp.transpose(W_sym, (1, 0, 2)).reshape(D_lat, K * S)
    b_symf = b_sym.reshape(1, K * S)
    b_in2 = b_in.reshape(1, D)
    w_sent2 = w_sent.reshape(D_lat, 1)
    Wg_flat = W_gate.reshape(K * S, D)
    Ws_flat = W_shift.reshape(K * S, D)
    bg_pad = jnp.zeros((16, D), dtype=b_gate.dtype).at[:K].set(b_gate)
    hopmask = (jnp.arange(_MAX_OPS_STATIC, dtype=jnp.int32)
               < jnp.asarray(max_ops, jnp.int32)).astype(jnp.int32)

    grid = (NT + 1,)
    body = functools.partial(_fused_body, num_nodes=K, sym_dim=S)
    prev = lambda i: (jnp.maximum(i - 1, 0), 0)
    cur = lambda i: (jnp.minimum(i, NT - 1), 0)
    out, pidx, sym_flat = pl.pallas_call(
        body,
        grid=grid,
        in_specs=[
            pl.BlockSpec((_TILE, D_in), cur),
            pl.BlockSpec((D_in, D), lambda i: (0, 0)),
            pl.BlockSpec((1, D), lambda i: (0, 0)),
            pl.BlockSpec((D_lat, K * S), lambda i: (0, 0)),
            pl.BlockSpec((1, K * S), lambda i: (0, 0)),
            pl.BlockSpec((D_lat, S), lambda i: (0, 0)),
            pl.BlockSpec((D_lat, 1), lambda i: (0, 0)),
            pl.BlockSpec((K * S, D), lambda i: (0, 0)),
            pl.BlockSpec((16, D), lambda i: (0, 0)),
            pl.BlockSpec((K * S, D), lambda i: (0, 0)),
            pl.BlockSpec(memory_space=pltpu.SMEM),
        ],
        out_specs=[
            pl.BlockSpec((_TILE, D), prev),
            pl.BlockSpec((_TILE, _MAX_OPS_STATIC), prev),
            pl.BlockSpec((_TILE, K * S), cur),
        ],
        out_shape=[
            jax.ShapeDtypeStruct((N, D), jnp.float32),
            jax.ShapeDtypeStruct((N, _MAX_OPS_STATIC), jnp.int32),
            jax.ShapeDtypeStruct((N, K * S), jnp.float32),
        ],
        scratch_shapes=[
            pltpu.VMEM((2, _TILE, D), jnp.float32),
            pltpu.VMEM((2, _TILE, K * S), jnp.float32),
        ],
    )(x, W_in, b_in2, W_symf, b_symf, W_q, w_sent2, Wg_flat, bg_pad, Ws_flat,
      hopmask)
    return out, pidx, sym_flat.reshape(N, K, S)
